# Initial kernel scaffold; baseline (speedup 1.0000x reference)
import jax
import jax.numpy as jnp
from jax.experimental import pallas as pl


def kernel(x, params, edge_index):
    raise NotImplementedError("write your pallas kernel here")


# speedup vs baseline: 31.8072x; 31.8072x over previous
import functools

import jax
import jax.numpy as jnp
from jax import lax
from jax.experimental import pallas as pl
from jax.experimental.pallas import tpu as pltpu
from jax.experimental.pallas import tpu_sc as plsc

_B = 8
_NPG = 1250
_EPG = 20000
_IN = 256
_HID = 512
_OUT = 40
_NP1 = 1280

_HI = lax.Precision.HIGHEST


_CH = 2000
_ROWS = 80
_TILE_W = _ROWS * _NP1
_GPC = 4


def _build_counts_sc(src, dst):
    mesh = plsc.VectorSubcoreMesh(core_axis_name="c", subcore_axis_name="s")

    @functools.partial(
        pl.kernel,
        out_type=jax.ShapeDtypeStruct((_B * _NP1 * _NP1,), jnp.float32),
        mesh=mesh,
        scratch_types=[
            pltpu.VMEM((_CH,), jnp.int32),
            pltpu.VMEM((_CH,), jnp.int32),
            pltpu.VMEM((_TILE_W,), jnp.float32),
        ],
        compiler_params=pltpu.CompilerParams(needs_layout_passes=False),
    )
    def kern(src_hbm, dst_hbm, out_hbm, src_v, dst_v, c_v):
        c = lax.axis_index("c")
        s = lax.axis_index("s")
        zeros16 = jnp.zeros((16,), jnp.float32)
        ones16 = jnp.ones((16,), jnp.float32)
        for gi in range(_GPC):
            g = c * _GPC + gi
            nbase = g * _NPG
            rbase = s * _ROWS

            def zbody(j, _):
                b = j * 128
                for u in range(8):
                    c_v[pl.ds(b + u * 16, 16)] = zeros16
                return 0

            lax.fori_loop(0, _TILE_W // 128, zbody, 0)

            for ch in range(_EPG // _CH):
                ebase = g * _EPG + ch * _CH
                pltpu.sync_copy(src_hbm.at[pl.ds(ebase, _CH)], src_v)
                pltpu.sync_copy(dst_hbm.at[pl.ds(ebase, _CH)], dst_v)

                def ebody(j, _):
                    sl = src_v[pl.ds(j * 16, 16)] - nbase
                    dl = dst_v[pl.ds(j * 16, 16)] - nbase - rbase
                    m = (dl >= 0) & (dl < _ROWS)
                    fidx = jnp.where(m, dl * _NP1 + sl, 0)
                    plsc.addupdate_scatter(c_v, [fidx], ones16, mask=m)
                    return 0

                lax.fori_loop(0, _CH // 16, ebody, 0)

            obase = g * (_NP1 * _NP1) + s * _TILE_W
            pltpu.sync_copy(c_v, out_hbm.at[pl.ds(obase, _TILE_W)])

    return kern(src, dst)


def _cumsum0(v):
    n = v.shape[0]
    d = 1
    while d < n:
        w = v.shape[1]
        v = v + jnp.concatenate(
            [jnp.zeros((d, w), jnp.float32), v[: n - d]], axis=0)
        d *= 2
    return v


def _split3(v):
    hi = v.astype(jnp.bfloat16)
    r1 = v - hi.astype(jnp.float32)
    mid = r1.astype(jnp.bfloat16)
    lo = (r1 - mid.astype(jnp.float32)).astype(jnp.bfloat16)
    return hi, mid, lo


def _cdot(Cb, v, dims=(((1,), (0,)), ((), ()))):
    hi, mid, lo = _split3(v)
    return (lax.dot_general(Cb, hi, dims, preferred_element_type=jnp.float32)
            + lax.dot_general(Cb, mid, dims,
                              preferred_element_type=jnp.float32)
            + lax.dot_general(Cb, lo, dims,
                              preferred_element_type=jnp.float32))


def _norms(Cb):
    npp = Cb.shape[0]
    ones = jnp.ones((npp, 1), jnp.bfloat16)
    deg_in = lax.dot_general(Cb, ones, (((1,), (0,)), ((), ())),
                             preferred_element_type=jnp.float32)
    deg_out = lax.dot_general(Cb, ones, (((0,), (0,)), ((), ())),
                              preferred_element_type=jnp.float32)
    cs = 1.0 / jnp.sqrt(jnp.maximum(deg_out, 1.0))
    cd = 1.0 / jnp.sqrt(jnp.maximum(deg_in, 1.0))
    return cs, cd


def _conv(Cb, cs, cd, v, W, b):
    agg = _cdot(Cb, v * cs) * cd
    return lax.dot_general(agg, W, (((1,), (0,)), ((), ())),
                           precision=_HI) + b


def _topk(Cb, h, Wsc, bsc, npg, k):
    npp = Cb.shape[0]
    cs, cd = _norms(Cb)
    v = lax.dot_general(h * cs, Wsc, (((1,), (0,)), ((), ())),
                        precision=_HI)
    score = _cdot(Cb, v) * cd + bsc
    row = lax.broadcasted_iota(jnp.int32, (npp, 1), 0)
    score = jnp.where(row < npg, score, -jnp.inf)

    km = lax.bitcast_convert_type(score, jnp.int32)
    km = jnp.where(km >= 0, km, km ^ jnp.int32(0x7FFFFFFF))
    ku = lax.bitcast_convert_type(km ^ jnp.int32(-2147483648), jnp.uint32)

    def tbody(i, t):
        cand = t | (jnp.uint32(1) << (jnp.uint32(31) - i.astype(jnp.uint32)))
        cnt = jnp.sum((ku >= cand).astype(jnp.int32))
        return jnp.where(cnt >= k, cand, t)

    T = lax.fori_loop(0, 32, tbody, jnp.uint32(0))

    gt = ku > T
    eq = ku == T
    n_gt = jnp.sum(gt.astype(jnp.int32))
    need = (k - n_gt).astype(jnp.float32)
    cums = _cumsum0(jnp.concatenate(
        [eq.astype(jnp.float32), gt.astype(jnp.float32)], axis=1))
    rank_eq = cums[:, 0:1] - eq.astype(jnp.float32)
    sel = gt | (eq & (rank_eq < need))

    gated = h * jnp.tanh(score)
    g = jnp.max(jnp.where(sel, gated, -jnp.inf), axis=0, keepdims=True)
    return gated, sel, score, g


def _chain(Cb, x, W1, b1, W2, b2, Wsc, bsc, npg, k):
    cs, cd = _norms(Cb)
    h = jax.nn.relu(_conv(Cb, cs, cd, x, W1, b1))
    h = jax.nn.relu(_conv(Cb, cs, cd, h, W2, b2))
    h = jax.nn.relu(_conv(Cb, cs, cd, h, W2, b2))
    return _topk(Cb, h, Wsc, bsc, npg, k)


def _conv2_body(C_ref, x_ref, W1_ref, b1_ref, W2_ref, b2_ref, h_ref):
    C = C_ref[0]
    cs, cd = _norms(C)
    h = jax.nn.relu(_conv(C, cs, cd, x_ref[0], W1_ref[...], b1_ref[...]))
    h_ref[0] = jax.nn.relu(_conv(C, cs, cd, h, W2_ref[...], b2_ref[...]))


def _conv1_body(C_ref, x_ref, W_ref, b_ref, h_ref):
    C = C_ref[0]
    cs, cd = _norms(C)
    h_ref[0] = jax.nn.relu(_conv(C, cs, cd, x_ref[0], W_ref[...], b_ref[...]))


def _score_body(C_ref, h_ref, Ws_ref, bs_ref, gated_ref, score_ref,
                sel_ref, *, npg, k):
    C = C_ref[0]
    gated, sel, score, _ = _topk(C, h_ref[0], Ws_ref[...], bs_ref[...],
                                 npg, k)
    gated_ref[0] = gated
    score_ref[0] = score
    sel_ref[0] = sel.astype(jnp.float32)


def _rank_body(sc_ref, sf_ref, sr_ref, fr_ref, rsel_ref):
    s_col = sc_ref[0]
    sf_col = sf_ref[0]
    npp = s_col.shape[0]
    ii = lax.broadcasted_iota(jnp.int32, (npp, 1), 0)
    ch = 256
    rank = jnp.zeros((npp, 1), jnp.float32)
    for c0 in range(0, npp, ch):
        w = min(ch, npp - c0)
        s_j = sr_ref[0, :, c0:c0 + w]
        sf_j = fr_ref[0, :, c0:c0 + w]
        jj = lax.broadcasted_iota(jnp.int32, (1, w), 1) + c0
        before = (s_j > s_col) | ((s_j == s_col) & (jj < ii))
        rank = rank + jnp.sum(jnp.where(before, sf_j, 0.0),
                              axis=1, keepdims=True)
    rsel_ref[0] = jnp.where(sf_col > 0.0, rank, -1.0)


def _pool_body(C_ref, rsel_ref, gated_ref, fn_ref, Cn_ref, g_ref, *, kpad):
    Cb = C_ref[0]
    rsel = rsel_ref[0]
    gated = gated_ref[0]
    npp = Cb.shape[0]
    kio = lax.broadcasted_iota(jnp.int32, (npp, kpad), 1).astype(jnp.float32)
    PTb = (rsel == kio).astype(jnp.bfloat16)
    tdims = (((0,), (0,)), ((), ()))
    fn_ref[0] = _cdot(PTb, gated, tdims)
    tmp = lax.dot_general(PTb, Cb, tdims,
                          preferred_element_type=jnp.float32)
    Cn_ref[0] = lax.dot_general(
        tmp.astype(jnp.bfloat16), PTb, (((1,), (0,)), ((), ())),
        preferred_element_type=jnp.float32).astype(jnp.bfloat16)
    g_ref[0] = jnp.max(jnp.where(rsel >= 0.0, gated, -jnp.inf),
                       axis=0, keepdims=True)


def _final_body(C_ref, x_ref, W1_ref, b1_ref, W2_ref, b2_ref, Ws_ref,
                bs_ref, g1_ref, g2_ref, l1W_ref, l1b_ref, l2W_ref, l2b_ref,
                l3W_ref, l3b_ref, out_ref, *, npg, k):
    C = C_ref[0]
    x = x_ref[0]
    _, _, _, g3 = _chain(
        C, x, W1_ref[...], b1_ref[...], W2_ref[...], b2_ref[...],
        Ws_ref[...], bs_ref[...], npg, k)
    gt = g1_ref[0] + g2_ref[0] + g3
    cat = jnp.concatenate([gt, gt], axis=1)
    f = jax.nn.relu(
        lax.dot_general(cat, l1W_ref[...], (((1,), (0,)), ((), ())),
                        precision=_HI) + l1b_ref[...])
    f = jax.nn.relu(
        lax.dot_general(f, l2W_ref[...], (((1,), (0,)), ((), ())),
                        precision=_HI) + l2b_ref[...])
    out_ref[0] = lax.dot_general(
        f, l3W_ref[...], (((1,), (0,)), ((), ())),
        precision=_HI) + l3b_ref[...]


def _full_spec(shp):
    nd = len(shp)
    return pl.BlockSpec(shp, lambda g, _n=nd: (0,) * _n)


def _block_call_split(C, feat, W1, b1, W2, b2, Wsc, bsc, *, npg, k, kpad,
                      interpret=False):
    bsz, npp, din = feat.shape
    hid = W2.shape[1]
    h2 = pl.pallas_call(
        _conv2_body,
        grid=(bsz,),
        in_specs=[
            pl.BlockSpec((1, npp, npp), lambda g: (g, 0, 0)),
            pl.BlockSpec((1, npp, din), lambda g: (g, 0, 0)),
            _full_spec(W1.shape), _full_spec(b1.shape),
            _full_spec(W2.shape), _full_spec(b2.shape),
        ],
        out_specs=pl.BlockSpec((1, npp, hid), lambda g: (g, 0, 0)),
        out_shape=jax.ShapeDtypeStruct((bsz, npp, hid), jnp.float32),
        interpret=interpret,
    )(C, feat, W1, b1, W2, b2)
    h3 = pl.pallas_call(
        _conv1_body,
        grid=(bsz,),
        in_specs=[
            pl.BlockSpec((1, npp, npp), lambda g: (g, 0, 0)),
            pl.BlockSpec((1, npp, hid), lambda g: (g, 0, 0)),
            _full_spec(W2.shape), _full_spec(b2.shape),
        ],
        out_specs=pl.BlockSpec((1, npp, hid), lambda g: (g, 0, 0)),
        out_shape=jax.ShapeDtypeStruct((bsz, npp, hid), jnp.float32),
        interpret=interpret,
    )(C, h2, W2, b2)
    gated, score, self_ = pl.pallas_call(
        functools.partial(_score_body, npg=npg, k=k),
        grid=(bsz,),
        in_specs=[
            pl.BlockSpec((1, npp, npp), lambda g: (g, 0, 0)),
            pl.BlockSpec((1, npp, hid), lambda g: (g, 0, 0)),
            _full_spec(Wsc.shape), _full_spec(bsc.shape),
        ],
        out_specs=[
            pl.BlockSpec((1, npp, hid), lambda g: (g, 0, 0)),
            pl.BlockSpec((1, npp, 1), lambda g: (g, 0, 0)),
            pl.BlockSpec((1, npp, 1), lambda g: (g, 0, 0)),
        ],
        out_shape=[
            jax.ShapeDtypeStruct((bsz, npp, hid), jnp.float32),
            jax.ShapeDtypeStruct((bsz, npp, 1), jnp.float32),
            jax.ShapeDtypeStruct((bsz, npp, 1), jnp.float32),
        ],
        interpret=interpret,
    )(C, h3, Wsc, bsc)
    rsel = pl.pallas_call(
        _rank_body,
        grid=(bsz,),
        in_specs=[
            pl.BlockSpec((1, npp, 1), lambda g: (g, 0, 0)),
            pl.BlockSpec((1, npp, 1), lambda g: (g, 0, 0)),
            pl.BlockSpec((1, 1, npp), lambda g: (g, 0, 0)),
            pl.BlockSpec((1, 1, npp), lambda g: (g, 0, 0)),
        ],
        out_specs=pl.BlockSpec((1, npp, 1), lambda g: (g, 0, 0)),
        out_shape=jax.ShapeDtypeStruct((bsz, npp, 1), jnp.float32),
        interpret=interpret,
    )(score, self_, score.reshape(bsz, 1, npp), self_.reshape(bsz, 1, npp))
    fn, Cn, g = pl.pallas_call(
        functools.partial(_pool_body, kpad=kpad),
        grid=(bsz,),
        in_specs=[
            pl.BlockSpec((1, npp, npp), lambda g: (g, 0, 0)),
            pl.BlockSpec((1, npp, 1), lambda g: (g, 0, 0)),
            pl.BlockSpec((1, npp, hid), lambda g: (g, 0, 0)),
        ],
        out_specs=[
            pl.BlockSpec((1, kpad, hid), lambda g: (g, 0, 0)),
            pl.BlockSpec((1, kpad, kpad), lambda g: (g, 0, 0)),
            pl.BlockSpec((1, 1, hid), lambda g: (g, 0, 0)),
        ],
        out_shape=[
            jax.ShapeDtypeStruct((bsz, kpad, hid), jnp.float32),
            jax.ShapeDtypeStruct((bsz, kpad, kpad), jnp.bfloat16),
            jax.ShapeDtypeStruct((bsz, 1, hid), jnp.float32),
        ],
        interpret=interpret,
    )(C, rsel, gated)
    return fn, Cn, g


def _final_call(C, feat, W1, b1, W2, b2, Wsc, bsc, g1, g2,
                l1W, l1b, l2W, l2b, l3W, l3b, *, npg, k, interpret=False):
    bsz, npp, din = feat.shape
    return pl.pallas_call(
        functools.partial(_final_body, npg=npg, k=k),
        grid=(bsz,),
        in_specs=[
            pl.BlockSpec((1, npp, npp), lambda g: (g, 0, 0)),
            pl.BlockSpec((1, npp, din), lambda g: (g, 0, 0)),
            _full_spec(W1.shape), _full_spec(b1.shape),
            _full_spec(W2.shape), _full_spec(b2.shape),
            _full_spec(Wsc.shape), _full_spec(bsc.shape),
            pl.BlockSpec((1, 1, _HID), lambda g: (g, 0, 0)),
            pl.BlockSpec((1, 1, _HID), lambda g: (g, 0, 0)),
            _full_spec(l1W.shape), _full_spec(l1b.shape),
            _full_spec(l2W.shape), _full_spec(l2b.shape),
            _full_spec(l3W.shape), _full_spec(l3b.shape),
        ],
        out_specs=pl.BlockSpec((1, 1, _OUT), lambda g: (g, 0, 0)),
        out_shape=jax.ShapeDtypeStruct((bsz, 1, _OUT), jnp.float32),
        interpret=interpret,
    )(C, feat, W1, b1, W2, b2, Wsc, bsc, g1, g2,
      l1W, l1b, l2W, l2b, l3W, l3b)


def _wprep(p):
    Wsc = (p["Ws1"] + p["Ws2"]) * 0.5
    bsc = ((p["bs1"] + p["bs2"]) * 0.5).reshape(1, 1)
    return (p["W1"], p["b1"].reshape(1, -1), p["W2"], p["b2"].reshape(1, -1),
            Wsc, bsc)


def _tc_forward(C1, xp, params, interpret=False):
    C1 = C1.astype(jnp.bfloat16)
    blocks = params["blocks"]
    f2, C2, g1 = _block_call_split(C1, xp, *_wprep(blocks[0]),
                                   npg=1250, k=625, kpad=640,
                                   interpret=interpret)
    f3, C3, g2 = _block_call_split(C2, f2, *_wprep(blocks[1]),
                                   npg=625, k=313, kpad=320,
                                   interpret=interpret)
    out = _final_call(
        C3, f3, *_wprep(blocks[2]), g1, g2,
        params["lin1_W"], params["lin1_b"].reshape(1, -1),
        params["lin2_W"], params["lin2_b"].reshape(1, -1),
        params["lin3_W"], params["lin3_b"].reshape(1, -1),
        npg=313, k=157, interpret=interpret)
    return out.reshape(_B, _OUT)


def kernel(x, params, edge_index):
    Cf = _build_counts_sc(edge_index[0], edge_index[1])
    C1 = Cf.reshape(_B, _NP1, _NP1)
    xp = jnp.pad(x.reshape(_B, _NPG, _IN), ((0, 0), (0, _NP1 - _NPG), (0, 0)))
    return _tc_forward(C1, xp, params)
